# Initial kernel scaffold; baseline (speedup 1.0000x reference)
import functools

import jax
import jax.numpy as jnp
from jax import lax
from jax.experimental import pallas as pl
from jax.experimental.pallas import tpu as pltpu
from jax.experimental.pallas import tpu_sc as plsc

BATCH = 16384
SEQ = 200
EMBED_DIM = 32
NUM_ROWS = BATCH * SEQ
G = 128
NGROUPS = NUM_ROWS // G


@functools.cache
def _build():
    info = plsc.get_sparse_core_info()
    nc, ns = info.num_cores, info.num_subcores
    nw = nc * ns
    groups_per_w = NGROUPS // nw
    J = 8
    CH = groups_per_w // J

    mesh = plsc.VectorSubcoreMesh(core_axis_name="c", subcore_axis_name="s")

    @functools.partial(
        pl.kernel,
        mesh=mesh,
        out_type=jax.ShapeDtypeStruct((NGROUPS, G, EMBED_DIM), jnp.float32),
        scratch_types=[
            pltpu.VMEM((J, G), jnp.int32),
            pltpu.VMEM((J, G, EMBED_DIM), jnp.float32),
            pltpu.SemaphoreType.DMA,
        ],
    )
    def k(idx_hbm, table_hbm, out_hbm, idx_v, rows_v, sem):
        wid = lax.axis_index("s") * nc + lax.axis_index("c")
        base = wid * groups_per_w

        def body(c, carry):
            gb = base + c * J
            pltpu.sync_copy(idx_hbm.at[pl.ds(gb, J)], idx_v)
            copies = [
                pltpu.async_copy(table_hbm.at[idx_v.at[j]], rows_v.at[j], sem)
                for j in range(J)
            ]
            for cp in copies:
                cp.wait()
            pltpu.sync_copy(rows_v, out_hbm.at[pl.ds(gb, J)])
            return carry

        lax.fori_loop(0, CH, body, 0)

    return k


def kernel(phoneme_ids, embeddings_weight):
    ids = phoneme_ids.reshape(NGROUPS, G).astype(jnp.int32)
    out = _build()(ids, embeddings_weight)
    return out.reshape(BATCH, SEQ, EMBED_DIM)


# speedup vs baseline: 6.1274x; 6.1274x over previous
import functools

import jax
import jax.numpy as jnp
from jax import lax
from jax.experimental import pallas as pl
from jax.experimental.pallas import tpu as pltpu
from jax.experimental.pallas import tpu_sc as plsc

BATCH = 16384
SEQ = 200
EMBED_DIM = 32
NUM_ROWS = BATCH * SEQ
G = 128
NGROUPS = NUM_ROWS // G


@functools.cache
def _build():
    info = plsc.get_sparse_core_info()
    nc, ns = info.num_cores, info.num_subcores
    nw = nc * ns
    groups_per_w = NGROUPS // nw
    J = 8
    CH = groups_per_w // J

    mesh = plsc.VectorSubcoreMesh(core_axis_name="c", subcore_axis_name="s")

    @functools.partial(
        pl.kernel,
        mesh=mesh,
        compiler_params=pltpu.CompilerParams(use_tc_tiling_on_sc=False),
        out_type=jax.ShapeDtypeStruct((NGROUPS, G, EMBED_DIM), jnp.float32),
        scratch_types=[
            pltpu.VMEM((J, G), jnp.int32),
            pltpu.VMEM((J, G, EMBED_DIM), jnp.float32),
            pltpu.SemaphoreType.DMA,
        ],
    )
    def k(idx_hbm, table_hbm, out_hbm, idx_v, rows_v, sem):
        wid = lax.axis_index("s") * nc + lax.axis_index("c")
        base = wid * groups_per_w

        def body(c, carry):
            gb = base + c * J
            pltpu.sync_copy(idx_hbm.at[pl.ds(gb, J)], idx_v)
            copies = [
                pltpu.async_copy(table_hbm.at[idx_v.at[j]], rows_v.at[j], sem)
                for j in range(J)
            ]
            for cp in copies:
                cp.wait()
            pltpu.sync_copy(rows_v, out_hbm.at[pl.ds(gb, J)])
            return carry

        lax.fori_loop(0, CH, body, 0)

    return k


def kernel(phoneme_ids, embeddings_weight):
    ids = phoneme_ids.reshape(NGROUPS, G).astype(jnp.int32)
    out = _build()(ids, embeddings_weight)
    return out.reshape(BATCH, SEQ, EMBED_DIM)


# speedup vs baseline: 6.1282x; 1.0001x over previous
import functools

import jax
import jax.numpy as jnp
from jax import lax
from jax.experimental import pallas as pl
from jax.experimental.pallas import tpu as pltpu
from jax.experimental.pallas import tpu_sc as plsc

BATCH = 16384
SEQ = 200
EMBED_DIM = 32
NUM_ROWS = BATCH * SEQ
G = 128
NGROUPS = NUM_ROWS // G


@functools.cache
def _build():
    info = plsc.get_sparse_core_info()
    nc, ns = info.num_cores, info.num_subcores
    nw = nc * ns
    groups_per_w = NGROUPS // nw
    J = 8
    CH = groups_per_w // J

    mesh = plsc.VectorSubcoreMesh(core_axis_name="c", subcore_axis_name="s")

    @functools.partial(
        pl.kernel,
        mesh=mesh,
        compiler_params=pltpu.CompilerParams(use_tc_tiling_on_sc=False),
        out_type=jax.ShapeDtypeStruct((NUM_ROWS, EMBED_DIM), jnp.float32),
        scratch_types=[
            pltpu.VMEM((J, G), jnp.int32),
            pltpu.VMEM((J * G, EMBED_DIM), jnp.float32),
            pltpu.SemaphoreType.DMA,
        ],
    )
    def k(idx_hbm, table_hbm, out_hbm, idx_v, rows_v, sem):
        wid = lax.axis_index("s") * nc + lax.axis_index("c")
        base = wid * groups_per_w

        def body(c, carry):
            gb = base + c * J
            pltpu.sync_copy(idx_hbm.at[pl.ds(gb, J)], idx_v)
            copies = [
                pltpu.async_copy(
                    table_hbm.at[idx_v.at[j]],
                    rows_v.at[pl.ds(j * G, G)],
                    sem,
                )
                for j in range(J)
            ]
            for cp in copies:
                cp.wait()
            pltpu.sync_copy(rows_v, out_hbm.at[pl.ds(gb * G, J * G)])
            return carry

        lax.fori_loop(0, CH, body, 0)

    return k


def kernel(phoneme_ids, embeddings_weight):
    ids = phoneme_ids.reshape(NGROUPS, G).astype(jnp.int32)
    out = _build()(ids, embeddings_weight)
    return out.reshape(BATCH, SEQ, EMBED_DIM)


# speedup vs baseline: 6.1903x; 1.0101x over previous
import functools

import jax
import jax.numpy as jnp
from jax import lax
from jax.experimental import pallas as pl
from jax.experimental.pallas import tpu as pltpu
from jax.experimental.pallas import tpu_sc as plsc

BATCH = 16384
SEQ = 200
EMBED_DIM = 32
G = 100
R = 8


@functools.cache
def _build():
    info = plsc.get_sparse_core_info()
    nc, ns = info.num_cores, info.num_subcores
    nw = nc * ns
    rows_per_w = BATCH // nw
    ch = rows_per_w // R

    mesh = plsc.VectorSubcoreMesh(core_axis_name="c", subcore_axis_name="s")

    @functools.partial(
        pl.kernel,
        mesh=mesh,
        compiler_params=pltpu.CompilerParams(use_tc_tiling_on_sc=False),
        out_type=jax.ShapeDtypeStruct((BATCH, SEQ, EMBED_DIM), jnp.float32),
        scratch_types=[
            pltpu.VMEM((2 * R, G), jnp.int32),
            pltpu.VMEM((R, SEQ, EMBED_DIM), jnp.float32),
            pltpu.SemaphoreType.DMA,
        ],
    )
    def k(idx_hbm, table_hbm, out_hbm, idx_v, rows_v, sem):
        wid = lax.axis_index("s") * nc + lax.axis_index("c")
        base = wid * rows_per_w

        def body(c, carry):
            rowb = base + c * R
            pltpu.sync_copy(idx_hbm.at[pl.ds(2 * rowb, 2 * R)], idx_v)
            copies = [
                pltpu.async_copy(
                    table_hbm.at[idx_v.at[2 * r + h]],
                    rows_v.at[r, pl.ds(h * G, G)],
                    sem,
                )
                for r in range(R)
                for h in range(2)
            ]
            for cp in copies:
                cp.wait()
            pltpu.sync_copy(rows_v, out_hbm.at[pl.ds(rowb, R)])
            return carry

        lax.fori_loop(0, ch, body, 0)

    return k


def kernel(phoneme_ids, embeddings_weight):
    ids = phoneme_ids.reshape(2 * BATCH, G).astype(jnp.int32)
    return _build()(ids, embeddings_weight)


# speedup vs baseline: 12.4580x; 2.0125x over previous
import functools

import jax
import jax.numpy as jnp
from jax import lax
from jax.experimental import pallas as pl
from jax.experimental.pallas import tpu as pltpu
from jax.experimental.pallas import tpu_sc as plsc

BATCH = 16384
SEQ = 200
EMBED_DIM = 32
G = 100
R = 8


@functools.cache
def _build():
    info = plsc.get_sparse_core_info()
    nc, ns = info.num_cores, info.num_subcores
    nw = nc * ns
    rows_per_w = BATCH // nw
    ch = rows_per_w // R

    mesh = plsc.VectorSubcoreMesh(core_axis_name="c", subcore_axis_name="s")

    @functools.partial(
        pl.kernel,
        mesh=mesh,
        compiler_params=pltpu.CompilerParams(use_tc_tiling_on_sc=False),
        out_type=jax.ShapeDtypeStruct((BATCH, SEQ, 128), jnp.float32),
        scratch_types=[
            pltpu.VMEM((2 * R, G), jnp.int32),
            pltpu.VMEM((R, SEQ, EMBED_DIM), jnp.float32),
            pltpu.SemaphoreType.DMA,
        ],
    )
    def k(idx_hbm, table_hbm, out_hbm, idx_v, rows_v, sem):
        wid = lax.axis_index("s") * nc + lax.axis_index("c")
        base = wid * rows_per_w

        def body(c, carry):
            rowb = base + c * R
            pltpu.sync_copy(idx_hbm.at[pl.ds(2 * rowb, 2 * R)], idx_v)
            copies = [
                pltpu.async_copy(
                    table_hbm.at[idx_v.at[2 * r + h]],
                    rows_v.at[r, pl.ds(h * G, G)],
                    sem,
                )
                for r in range(R)
                for h in range(2)
            ]
            for cp in copies:
                cp.wait()
            pltpu.sync_copy(
                rows_v, out_hbm.at[pl.ds(rowb, R), :, pl.ds(0, EMBED_DIM)]
            )
            return carry

        lax.fori_loop(0, ch, body, 0)

    return k


def kernel(phoneme_ids, embeddings_weight):
    ids = phoneme_ids.reshape(2 * BATCH, G).astype(jnp.int32)
    out_padded = _build()(ids, embeddings_weight)
    return out_padded[:, :, :EMBED_DIM]
